# baseline (speedup 1.0000x reference)
import jax
import jax.numpy as jnp
from jax.experimental import pallas as pl
from jax.experimental.pallas import tpu as pltpu

ALPHA_ = 0.5
HB = 16


def _midmax_body(x_ref, o_ref):
    v = x_ref[0].reshape(HB, 2, 128, 128)
    h0 = v[:, 0]
    h1 = v[:, 1]
    vmin = jnp.minimum(h0, h1)
    vmax = jnp.maximum(h0, h1)
    m1 = vmin[:, :, :64]
    m2 = vmin[:, :, 64:]
    M1 = vmax[:, :, :64]
    M2 = vmax[:, :, 64:]
    max4 = jnp.maximum(M1, M2)
    sec = jnp.minimum(jnp.maximum(m1, m2), jnp.minimum(M1, M2))
    res = ALPHA_ * max4 + (1.0 - ALPHA_) * jnp.maximum(sec, 0.0)
    r4 = res.reshape(HB, 64, 2, 64)
    o_ref[0] = jnp.concatenate([r4[:, :, 0, :], r4[:, :, 1, :]], axis=-1)


NCHUNK = 2


def kernel(x):
    B, H, W, C = x.shape
    Ho, Wo = H // 2, W // 2
    cb = B // NCHUNK
    outs = []
    for i in range(NCHUNK):
        xi = x[i * cb:(i + 1) * cb].reshape(cb, H, (W * C) // 128, 128)
        outs.append(pl.pallas_call(
            _midmax_body,
            grid=(cb, Ho // HB),
            in_specs=[pl.BlockSpec((1, 2 * HB, (W * C) // 128, 128),
                                   lambda b, h: (b, h, 0, 0))],
            out_specs=pl.BlockSpec((1, HB, (Wo * C) // 128, 128),
                                   lambda b, h: (b, h, 0, 0)),
            out_shape=jax.ShapeDtypeStruct((cb, Ho, (Wo * C) // 128, 128),
                                           x.dtype),
            compiler_params=pltpu.CompilerParams(
                dimension_semantics=("parallel", "arbitrary")),
        )(xi))
    return jnp.concatenate(outs, axis=0).reshape(B, Ho, Wo, C)


# speedup vs baseline: 1.3105x; 1.3105x over previous
import jax
import jax.numpy as jnp
from jax.experimental import pallas as pl
from jax.experimental.pallas import tpu as pltpu

ALPHA_ = 0.5
HB = 16


def _midmax_body(x_ref, o_ref):
    v = x_ref[0].reshape(HB, 2, 128, 128)
    h0 = v[:, 0]
    h1 = v[:, 1]
    vmin = jnp.minimum(h0, h1)
    vmax = jnp.maximum(h0, h1)
    m1 = vmin[:, :, :64]
    m2 = vmin[:, :, 64:]
    M1 = vmax[:, :, :64]
    M2 = vmax[:, :, 64:]
    max4 = jnp.maximum(M1, M2)
    sec = jnp.minimum(jnp.maximum(m1, m2), jnp.minimum(M1, M2))
    res = ALPHA_ * max4 + (1.0 - ALPHA_) * jnp.maximum(sec, 0.0)
    r4 = res.reshape(HB, 64, 2, 64)
    o_ref[0] = jnp.concatenate([r4[:, :, 0, :], r4[:, :, 1, :]], axis=-1)


def kernel(x):
    B, H, W, C = x.shape
    Ho, Wo = H // 2, W // 2
    xr = x.reshape(B, H, (W * C) // 128, 128)
    grid = (B, Ho // HB)
    out5 = pl.pallas_call(
        _midmax_body,
        grid=grid,
        in_specs=[pl.BlockSpec((1, 2 * HB, (W * C) // 128, 128),
                               lambda b, h: (b, h, 0, 0))],
        out_specs=pl.BlockSpec((1, HB, (Wo * C) // 128, 128),
                               lambda b, h: (b, h, 0, 0)),
        out_shape=jax.ShapeDtypeStruct((B, Ho, (Wo * C) // 128, 128), x.dtype),
        compiler_params=pltpu.CompilerParams(
            dimension_semantics=("parallel", "arbitrary")),
    )(xr)
    return out5.reshape(B, Ho, Wo, C)


# speedup vs baseline: 1.3823x; 1.0549x over previous
import jax
import jax.numpy as jnp
from jax.experimental import pallas as pl
from jax.experimental.pallas import tpu as pltpu

ALPHA_ = 0.5
HB = 32


def _midmax_body(x_ref, o_ref):
    v = x_ref[0].reshape(HB, 2, 128, 128)
    h0 = v[:, 0]
    h1 = v[:, 1]
    vmin = jnp.minimum(h0, h1)
    vmax = jnp.maximum(h0, h1)
    m1 = vmin[:, :, :64]
    m2 = vmin[:, :, 64:]
    M1 = vmax[:, :, :64]
    M2 = vmax[:, :, 64:]
    max4 = jnp.maximum(M1, M2)
    sec = jnp.minimum(jnp.maximum(m1, m2), jnp.minimum(M1, M2))
    res = ALPHA_ * max4 + (1.0 - ALPHA_) * jnp.maximum(sec, 0.0)
    r4 = res.reshape(HB, 64, 2, 64)
    o_ref[0] = jnp.concatenate([r4[:, :, 0, :], r4[:, :, 1, :]], axis=-1)


def kernel(x):
    B, H, W, C = x.shape
    Ho, Wo = H // 2, W // 2
    xr = x.reshape(B, H, (W * C) // 128, 128)
    grid = (B, Ho // HB)
    out5 = pl.pallas_call(
        _midmax_body,
        grid=grid,
        in_specs=[pl.BlockSpec((1, 2 * HB, (W * C) // 128, 128),
                               lambda b, h: (b, h, 0, 0))],
        out_specs=pl.BlockSpec((1, HB, (Wo * C) // 128, 128),
                               lambda b, h: (b, h, 0, 0)),
        out_shape=jax.ShapeDtypeStruct((B, Ho, (Wo * C) // 128, 128), x.dtype),
        compiler_params=pltpu.CompilerParams(
            dimension_semantics=("parallel", "arbitrary")),
    )(xr)
    return out5.reshape(B, Ho, Wo, C)


# speedup vs baseline: 1.3861x; 1.0027x over previous
import jax
import jax.numpy as jnp
from jax.experimental import pallas as pl
from jax.experimental.pallas import tpu as pltpu

ALPHA_ = 0.5
HB = 64


def _midmax_body(x_ref, o_ref):
    v = x_ref[0].reshape(HB, 2, 128, 128)
    h0 = v[:, 0]
    h1 = v[:, 1]
    vmin = jnp.minimum(h0, h1)
    vmax = jnp.maximum(h0, h1)
    m1 = vmin[:, :, :64]
    m2 = vmin[:, :, 64:]
    M1 = vmax[:, :, :64]
    M2 = vmax[:, :, 64:]
    max4 = jnp.maximum(M1, M2)
    sec = jnp.minimum(jnp.maximum(m1, m2), jnp.minimum(M1, M2))
    res = ALPHA_ * max4 + (1.0 - ALPHA_) * jnp.maximum(sec, 0.0)
    r4 = res.reshape(HB, 64, 2, 64)
    o_ref[0] = jnp.concatenate([r4[:, :, 0, :], r4[:, :, 1, :]], axis=-1)


def kernel(x):
    B, H, W, C = x.shape
    Ho, Wo = H // 2, W // 2
    xr = x.reshape(B, H, (W * C) // 128, 128)
    grid = (B, Ho // HB)
    out5 = pl.pallas_call(
        _midmax_body,
        grid=grid,
        in_specs=[pl.BlockSpec((1, 2 * HB, (W * C) // 128, 128),
                               lambda b, h: (b, h, 0, 0))],
        out_specs=pl.BlockSpec((1, HB, (Wo * C) // 128, 128),
                               lambda b, h: (b, h, 0, 0)),
        out_shape=jax.ShapeDtypeStruct((B, Ho, (Wo * C) // 128, 128), x.dtype),
        compiler_params=pltpu.CompilerParams(
            dimension_semantics=("parallel", "arbitrary")),
    )(xr)
    return out5.reshape(B, Ho, Wo, C)


# speedup vs baseline: 1.3875x; 1.0011x over previous
import jax
import jax.numpy as jnp
from jax.experimental import pallas as pl
from jax.experimental.pallas import tpu as pltpu

ALPHA_ = 0.5
HB = 64


def _midmax_body(x_ref, o_ref):
    v = x_ref[0].reshape(HB, 2, 128, 128)
    h0 = v[:, 0]
    h1 = v[:, 1]
    vmin = jnp.minimum(h0, h1)
    vmax = jnp.maximum(h0, h1)
    m1 = vmin[:, :, :64]
    m2 = vmin[:, :, 64:]
    M1 = vmax[:, :, :64]
    M2 = vmax[:, :, 64:]
    max4 = jnp.maximum(M1, M2)
    sec = jnp.minimum(jnp.maximum(m1, m2), jnp.minimum(M1, M2))
    res = ALPHA_ * max4 + (1.0 - ALPHA_) * jnp.maximum(sec, 0.0)
    r4 = res.reshape(HB, 64, 2, 64)
    o_ref[0] = jnp.concatenate([r4[:, :, 0, :], r4[:, :, 1, :]], axis=-1)


def kernel(x):
    B, H, W, C = x.shape
    Ho, Wo = H // 2, W // 2
    xr = x.reshape(B, H, (W * C) // 128, 128)
    grid = (B, Ho // HB)
    out5 = pl.pallas_call(
        _midmax_body,
        grid=grid,
        in_specs=[pl.BlockSpec((1, 2 * HB, (W * C) // 128, 128),
                               lambda b, h: (b, h, 0, 0))],
        out_specs=pl.BlockSpec((1, HB, (Wo * C) // 128, 128),
                               lambda b, h: (b, h, 0, 0)),
        out_shape=jax.ShapeDtypeStruct((B, Ho, (Wo * C) // 128, 128), x.dtype),
        compiler_params=pltpu.CompilerParams(
            dimension_semantics=("parallel", "parallel")),
    )(xr)
    return out5.reshape(B, Ho, Wo, C)
